# baseline (speedup 1.0000x reference)
import functools

import jax
import jax.numpy as jnp
from jax import lax
from jax.experimental import pallas as pl
from jax.experimental.pallas import tpu as pltpu
from jax.experimental.pallas import tpu_sc as plsc

N = 10000
E = 320000
B = 128
H = 64
NS = 16
CHUNK = 80
EPW = E // NS
NCHUNK = EPW // CHUNK
GAT_W = 80
NBUF = 2
ZROW = 40
SLAB = 125
NSLAB = NCHUNK // SLAB

f32 = jnp.float32
i32 = jnp.int32


def _tile_rows(s):
    start = s * 640
    nz = jnp.minimum(16, (N - start) // ZROW)
    return start, nz


def _zero_vmem(zb, nrow, ncol):
    zeros16 = jnp.zeros((16,), f32)

    def body(t, _):
        for j in range(ncol // 16):
            zb[t, pl.ds(j * 16, 16)] = zeros16
        return ()
    lax.fori_loop(0, nrow, body, ())


def _sc_gat_body(h2, as2, ad2, src4, dst4, acc_out,
                 asv, adv, sidx, didx, buf1a, buf1b, buf2,
                 exv, zb, accs, sema, semb):
    c = lax.axis_index("c")
    s = lax.axis_index("s")

    _zero_vmem(zb, ZROW, GAT_W)
    start, nz = _tile_rows(s)

    def zbody(t, _):
        pltpu.sync_copy(zb, accs.at[pl.ds(start + t * ZROW, ZROW)])
        return ()
    lax.fori_loop(0, nz, zbody, ())

    pltpu.sync_copy(as2, asv)
    pltpu.sync_copy(ad2, adv)
    plsc.subcore_barrier()

    off = c * N
    bufs1 = (buf1a, buf1b)
    sems = (sema, semb)

    def _weight_chunk(t, b1):
        for gg in range(CHUNK // 16):
            si = sidx[t, pl.ds(gg * 16, 16)]
            di = didx[t, pl.ds(gg * 16, 16)] + off
            e = plsc.load_gather(asv, [si]) + plsc.load_gather(adv, [di])
            e = jnp.maximum(e, 0.2 * e)
            exv[pl.ds(gg * 16, 16)] = e

        def scale_body(u, _):
            for r in range(10):
                i = u * 10 + r
                w = plsc.load_gather(exv, [jnp.zeros((16,), i32) + i])
                for j in range(H // 16):
                    buf2[i, pl.ds(j * 16, 16)] = b1[i, pl.ds(j * 16, 16)] * w
                buf2[i, pl.ds(H, 16)] = w
            return ()
        lax.fori_loop(0, CHUNK // 10, scale_body, ())

    for k in range(NSLAB):
        pltpu.sync_copy(src4.at[c, s, pl.ds(k * SLAB, SLAB)], sidx)
        pltpu.sync_copy(dst4.at[c, s, pl.ds(k * SLAB, SLAB)], didx)

        for b in range(NBUF):
            pltpu.make_async_copy(h2.at[sidx.at[b]], bufs1[b], sems[b]).start()

        def group_body(g, _):
            for b in range(NBUF):
                t = g * NBUF + b
                pltpu.make_async_copy(
                    h2.at[sidx.at[t]], bufs1[b], sems[b]).wait()
                _weight_chunk(t, bufs1[b])
                nxt = t + NBUF

                @pl.when(nxt < SLAB)
                def _():
                    pltpu.make_async_copy(
                        h2.at[sidx.at[nxt]], bufs1[b], sems[b]).start()

                pltpu.sync_copy(buf2, accs.at[didx.at[t]], add=True)
            return ()
        lax.fori_loop(0, SLAB // NBUF, group_body, ())

        for t in range(SLAB - SLAB % NBUF, SLAB):
            b = t % NBUF
            pltpu.make_async_copy(h2.at[sidx.at[t]], bufs1[b], sems[b]).wait()
            _weight_chunk(t, bufs1[b])
            pltpu.sync_copy(buf2, accs.at[didx.at[t]], add=True)

    plsc.subcore_barrier()

    def wbody(t, _):
        r = start + t * ZROW
        pltpu.sync_copy(accs.at[pl.ds(r, ZROW)], acc_out.at[c, pl.ds(r, ZROW)])
        return ()
    lax.fori_loop(0, nz, wbody, ())


_sc_gat = pl.kernel(
    _sc_gat_body,
    out_type=jax.ShapeDtypeStruct((2, N, GAT_W), f32),
    mesh=plsc.VectorSubcoreMesh(core_axis_name="c", subcore_axis_name="s"),
    scratch_types=[
        pltpu.VMEM((2 * N,), f32),
        pltpu.VMEM((2 * N,), f32),
        pltpu.VMEM((SLAB, CHUNK), i32),
        pltpu.VMEM((SLAB, CHUNK), i32),
        pltpu.VMEM((CHUNK, H), f32),
        pltpu.VMEM((CHUNK, H), f32),
        pltpu.VMEM((CHUNK, GAT_W), f32),
        pltpu.VMEM((CHUNK,), f32),
        pltpu.VMEM((ZROW, GAT_W), f32),
        pltpu.VMEM_SHARED((N, GAT_W), f32),
        pltpu.SemaphoreType.DMA,
        pltpu.SemaphoreType.DMA,
    ],
    compiler_params=pltpu.CompilerParams(needs_layout_passes=False,
                                         use_tc_tiling_on_sc=False),
)


def _sc_gin_body(x2, src4, dst4, agg_out,
                 sidx, didx, bufa, bufb, zb, accs, sema, semb):
    c = lax.axis_index("c")
    s = lax.axis_index("s")

    _zero_vmem(zb, ZROW, H)
    start, nz = _tile_rows(s)

    def zbody(t, _):
        pltpu.sync_copy(zb, accs.at[pl.ds(start + t * ZROW, ZROW)])
        return ()
    lax.fori_loop(0, nz, zbody, ())

    pltpu.sync_copy(src4.at[c, s], sidx)
    pltpu.sync_copy(dst4.at[c, s], didx)
    plsc.subcore_barrier()

    bufs = (bufa, bufb)
    sems = (sema, semb)

    for b in range(NBUF):
        pltpu.make_async_copy(x2.at[sidx.at[b]], bufs[b], sems[b]).start()

    def group_body(g, _):
        for b in range(NBUF):
            t = g * NBUF + b
            pltpu.make_async_copy(x2.at[sidx.at[t]], bufs[b], sems[b]).wait()
            nxt = t + NBUF

            @pl.when(nxt < NCHUNK)
            def _():
                pltpu.make_async_copy(
                    x2.at[sidx.at[nxt]], bufs[b], sems[b]).start()

            pltpu.sync_copy(bufs[b], accs.at[didx.at[t]], add=True)
        return ()
    lax.fori_loop(0, NCHUNK // NBUF, group_body, ())

    plsc.subcore_barrier()

    def wbody(t, _):
        r = start + t * ZROW
        pltpu.sync_copy(accs.at[pl.ds(r, ZROW)], agg_out.at[c, pl.ds(r, ZROW)])
        return ()
    lax.fori_loop(0, nz, wbody, ())


_sc_gin = pl.kernel(
    _sc_gin_body,
    out_type=jax.ShapeDtypeStruct((2, N, H), f32),
    mesh=plsc.VectorSubcoreMesh(core_axis_name="c", subcore_axis_name="s"),
    scratch_types=[
        pltpu.VMEM((NCHUNK, CHUNK), i32),
        pltpu.VMEM((NCHUNK, CHUNK), i32),
        pltpu.VMEM((CHUNK, H), f32),
        pltpu.VMEM((CHUNK, H), f32),
        pltpu.VMEM((ZROW, H), f32),
        pltpu.VMEM_SHARED((N, H), f32),
        pltpu.SemaphoreType.DMA,
        pltpu.SemaphoreType.DMA,
    ],
    compiler_params=pltpu.CompilerParams(use_tc_tiling_on_sc=False),
)


POOL_W = 3 * H
POOL_CHUNKS = N // CHUNK


def _sc_pool_body(gcat, batch, pool_out, bidx, buf, zb, accs, sem):
    c = lax.axis_index("c")
    s = lax.axis_index("s")

    _zero_vmem(zb, 8, POOL_W)
    pltpu.sync_copy(zb, accs.at[pl.ds(s * 8, 8)])
    plsc.subcore_barrier()

    def chunk_body(k, _):
        ch = s + k * NS

        @pl.when(ch < POOL_CHUNKS)
        def _():
            base = ch * CHUNK
            pltpu.sync_copy(batch.at[pl.ds(base, CHUNK)], bidx)
            pltpu.sync_copy(gcat.at[c, pl.ds(base, CHUNK)], buf)
            pltpu.sync_copy(buf, accs.at[bidx], add=True)
        return ()
    lax.fori_loop(0, (POOL_CHUNKS + NS - 1) // NS, chunk_body, ())

    plsc.subcore_barrier()
    pltpu.sync_copy(accs.at[pl.ds(s * 8, 8)],
                    pool_out.at[c, pl.ds(s * 8, 8)])


_sc_pool = pl.kernel(
    _sc_pool_body,
    out_type=jax.ShapeDtypeStruct((2, B, POOL_W), f32),
    mesh=plsc.VectorSubcoreMesh(core_axis_name="c", subcore_axis_name="s"),
    scratch_types=[
        pltpu.VMEM((CHUNK,), i32),
        pltpu.VMEM((CHUNK, POOL_W), f32),
        pltpu.VMEM((8, POOL_W), f32),
        pltpu.VMEM_SHARED((B, POOL_W), f32),
        pltpu.SemaphoreType.DMA,
    ],
    compiler_params=pltpu.CompilerParams(use_tc_tiling_on_sc=False),
)


def _bn(x, g, b, eps=1e-5):
    m = jnp.mean(x, axis=0)
    v = jnp.mean((x - m) ** 2, axis=0)
    return (x - m) / jnp.sqrt(v + eps) * g + b


def _tc_prep_body(xr, xp, wr, wp, avr, avp, h_ref, a_ref):
    for idx, (x, w, av) in enumerate(((xr, wr, avr), (xp, wp, avp))):
        h = jnp.dot(x[...], w[...], preferred_element_type=f32)
        h_ref[idx] = h
        a_s = jnp.sum(h * av[0], axis=1)
        a_d = jnp.sum(h * av[1], axis=1)
        a_ref[idx] = jnp.concatenate(
            [a_s[:, None], a_d[:, None], jnp.zeros((N, 6), f32)], axis=1)


_TC_PARAMS = pltpu.CompilerParams(vmem_limit_bytes=110 * 1024 * 1024)

_tc_prep = pl.pallas_call(
    _tc_prep_body,
    out_shape=[jax.ShapeDtypeStruct((2, N, H), f32),
               jax.ShapeDtypeStruct((2, N, 8), f32)],
    compiler_params=_TC_PARAMS,
)


def _tc_gatpost_body(acc, br, bp, gr, cbr, gp, cbp, g1_ref):
    for idx, (bb, g, cb) in enumerate(((br, gr, cbr), (bp, gp, cbp))):
        num = acc[idx, :, :H]
        den = acc[idx, :, H:H + 1]
        y = num / (den + 1e-16) + bb[...]
        g1_ref[idx] = jax.nn.relu(_bn(y, g[...], cb[...]))


_tc_gatpost = pl.pallas_call(
    _tc_gatpost_body,
    out_shape=jax.ShapeDtypeStruct((2, N, H), f32),
    compiler_params=_TC_PARAMS,
)


def _gin_mlp(x, agg, w1, b1, g1, be1, w2, b2, g2, be2):
    h = x + agg
    h = jnp.dot(h, w1[...], preferred_element_type=f32) + b1[...]
    h = jax.nn.relu(_bn(h, g1[...], be1[...]))
    h = jnp.dot(h, w2[...], preferred_element_type=f32) + b2[...]
    return jax.nn.relu(_bn(h, g2[...], be2[...]))


def _tc_gin_body(x, agg, *args):
    out_ref = args[-1]
    pr, pp = args[:8], args[8:16]
    for idx, p in enumerate((pr, pp)):
        out_ref[idx] = _gin_mlp(x[idx], agg[idx], *p)


_tc_gin = pl.pallas_call(
    _tc_gin_body,
    out_shape=jax.ShapeDtypeStruct((2, N, H), f32),
    compiler_params=_TC_PARAMS,
)


def _tc_gin_cat_body(x, agg, g1, *args):
    out_ref = args[-1]
    pr, pp = args[:8], args[8:16]
    for idx, p in enumerate((pr, pp)):
        g3 = _gin_mlp(x[idx], agg[idx], *p)
        out_ref[idx] = jnp.concatenate([g1[idx], x[idx], g3], axis=1)


_tc_gin_cat = pl.pallas_call(
    _tc_gin_cat_body,
    out_shape=jax.ShapeDtypeStruct((2, N, POOL_W), f32),
    compiler_params=_TC_PARAMS,
)


def _tc_head_body(pool, w1, b1, w2, b2, wc, bc, out_ref):
    conv = jnp.concatenate([pool[0], pool[1]], axis=1)
    z = jnp.dot(conv, w1[...], preferred_element_type=f32) + b1[...]
    z = jnp.dot(z, w2[...], preferred_element_type=f32) + b2[...]
    z = jax.nn.relu(z)
    out_ref[...] = jnp.dot(z, wc[...], preferred_element_type=f32) + bc[...]


_tc_head = pl.pallas_call(
    _tc_head_body,
    out_shape=jax.ShapeDtypeStruct((B, 10), f32),
    compiler_params=_TC_PARAMS,
)


def _gin_args(p):
    return (p["W1"], p["b1"][None, :], p["g1"][None, :], p["be1"][None, :],
            p["W2"], p["b2"][None, :], p["g2"][None, :], p["be2"][None, :])


def kernel(x_r, edge_r, x_p, edge_p, batch, params):
    pr, pp = params["gat_r"], params["gat_p"]
    src = jnp.concatenate([edge_r[0], edge_p[0] + N]).reshape(
        2, NS, NCHUNK, CHUNK)
    dst = jnp.concatenate([edge_r[1], edge_p[1]]).reshape(
        2, NS, NCHUNK, CHUNK)

    h, a8 = _tc_prep(x_r, x_p, pr["W"], pp["W"],
                     jnp.stack([pr["asrc"], pr["adst"]]),
                     jnp.stack([pp["asrc"], pp["adst"]]))
    h2 = h.reshape(2 * N, H)
    a2 = a8.reshape(2 * N, 8)
    acc = _sc_gat(h2, a2[:, 0], a2[:, 1], src, dst)

    g1 = _tc_gatpost(acc,
                     pr["b"][None, :], pp["b"][None, :],
                     params["bn_r"]["g"][None, :], params["bn_r"]["b"][None, :],
                     params["bn_p"]["g"][None, :], params["bn_p"]["b"][None, :])

    agg1 = _sc_gin(g1.reshape(2 * N, H), src, dst)
    g2 = _tc_gin(g1, agg1, *_gin_args(params["gin2_r"]),
                 *_gin_args(params["gin2_p"]))
    agg2 = _sc_gin(g2.reshape(2 * N, H), src, dst)
    gcat = _tc_gin_cat(g2, agg2, g1, *_gin_args(params["gin3_r"]),
                       *_gin_args(params["gin3_p"]))

    pool = _sc_pool(gcat, batch)

    fc = params["fc"]
    return _tc_head(pool, fc["W1"], fc["b1"][None, :],
                    fc["W2"], fc["b2"][None, :],
                    fc["Wc"], fc["bc"][None, :])


# speedup vs baseline: 1.4368x; 1.4368x over previous
import functools

import jax
import jax.numpy as jnp
from jax import lax
from jax.experimental import pallas as pl
from jax.experimental.pallas import tpu as pltpu
from jax.experimental.pallas import tpu_sc as plsc

N = 10000
E = 320000
B = 128
H = 64
NS = 16
CHUNK = 80
EPW = E // NS
NCHUNK = EPW // CHUNK
GAT_W = 80
NBUF = 2
ZROW = 40
SLAB = 125
NSLAB = NCHUNK // SLAB

f32 = jnp.float32
i32 = jnp.int32


def _tile_rows(s):
    start = s * 640
    nz = jnp.minimum(16, (N - start) // ZROW)
    return start, nz


def _zero_vmem(zb, nrow, ncol):
    zeros16 = jnp.zeros((16,), f32)

    def body(t, _):
        for j in range(ncol // 16):
            zb[t, pl.ds(j * 16, 16)] = zeros16
        return ()
    lax.fori_loop(0, nrow, body, ())


def _sc_gat_body(h2, as2, ad2, src4, dst4, acc_out,
                 asv, adv, sidx, didx, buf1a, buf1b, buf2,
                 exv, zb, accs, sema, semb):
    c = lax.axis_index("c")
    s = lax.axis_index("s")

    _zero_vmem(zb, ZROW, GAT_W)
    start, nz = _tile_rows(s)

    def zbody(t, _):
        pltpu.sync_copy(zb, accs.at[pl.ds(start + t * ZROW, ZROW)])
        return ()
    lax.fori_loop(0, nz, zbody, ())

    pltpu.sync_copy(as2, asv)
    pltpu.sync_copy(ad2, adv)
    plsc.subcore_barrier()

    off = c * N
    bufs1 = (buf1a, buf1b)
    sems = (sema, semb)

    def _weight_chunk(t, b1):
        for gg in range(CHUNK // 16):
            si = sidx[t, pl.ds(gg * 16, 16)]
            di = didx[t, pl.ds(gg * 16, 16)] + off
            e = plsc.load_gather(asv, [si]) + plsc.load_gather(adv, [di])
            e = jnp.maximum(e, 0.2 * e)
            exv[pl.ds(gg * 16, 16)] = jnp.exp(e)

        @plsc.parallel_loop(0, CHUNK, 1, unroll=10)
        def scale_body(i):
            w = plsc.load_gather(exv, [jnp.zeros((16,), i32) + i])
            for j in range(H // 16):
                buf2[i, pl.ds(j * 16, 16)] = b1[i, pl.ds(j * 16, 16)] * w
            buf2[i, pl.ds(H, 16)] = w

    for k in range(NSLAB):
        pltpu.sync_copy(src4.at[c, s, pl.ds(k * SLAB, SLAB)], sidx)
        pltpu.sync_copy(dst4.at[c, s, pl.ds(k * SLAB, SLAB)], didx)

        for b in range(NBUF):
            pltpu.make_async_copy(h2.at[sidx.at[b]], bufs1[b], sems[b]).start()

        def group_body(g, _):
            for b in range(NBUF):
                t = g * NBUF + b
                pltpu.make_async_copy(
                    h2.at[sidx.at[t]], bufs1[b], sems[b]).wait()
                _weight_chunk(t, bufs1[b])
                nxt = t + NBUF

                @pl.when(nxt < SLAB)
                def _():
                    pltpu.make_async_copy(
                        h2.at[sidx.at[nxt]], bufs1[b], sems[b]).start()

                pltpu.sync_copy(buf2, accs.at[didx.at[t]], add=True)
            return ()
        lax.fori_loop(0, SLAB // NBUF, group_body, ())

        for t in range(SLAB - SLAB % NBUF, SLAB):
            b = t % NBUF
            pltpu.make_async_copy(h2.at[sidx.at[t]], bufs1[b], sems[b]).wait()
            _weight_chunk(t, bufs1[b])
            pltpu.sync_copy(buf2, accs.at[didx.at[t]], add=True)

    plsc.subcore_barrier()

    def wbody(t, _):
        r = start + t * ZROW
        pltpu.sync_copy(accs.at[pl.ds(r, ZROW)], acc_out.at[c, pl.ds(r, ZROW)])
        return ()
    lax.fori_loop(0, nz, wbody, ())


_sc_gat = pl.kernel(
    _sc_gat_body,
    out_type=jax.ShapeDtypeStruct((2, N, GAT_W), f32),
    mesh=plsc.VectorSubcoreMesh(core_axis_name="c", subcore_axis_name="s"),
    scratch_types=[
        pltpu.VMEM((2 * N,), f32),
        pltpu.VMEM((2 * N,), f32),
        pltpu.VMEM((SLAB, CHUNK), i32),
        pltpu.VMEM((SLAB, CHUNK), i32),
        pltpu.VMEM((CHUNK, H), f32),
        pltpu.VMEM((CHUNK, H), f32),
        pltpu.VMEM((CHUNK, GAT_W), f32),
        pltpu.VMEM((CHUNK,), f32),
        pltpu.VMEM((ZROW, GAT_W), f32),
        pltpu.VMEM_SHARED((N, GAT_W), f32),
        pltpu.SemaphoreType.DMA,
        pltpu.SemaphoreType.DMA,
    ],
    compiler_params=pltpu.CompilerParams(needs_layout_passes=False,
                                         use_tc_tiling_on_sc=False),
)


def _sc_gin_body(x2, src4, dst4, agg_out,
                 sidx, didx, bufa, bufb, zb, accs, sema, semb):
    c = lax.axis_index("c")
    s = lax.axis_index("s")

    _zero_vmem(zb, ZROW, H)
    start, nz = _tile_rows(s)

    def zbody(t, _):
        pltpu.sync_copy(zb, accs.at[pl.ds(start + t * ZROW, ZROW)])
        return ()
    lax.fori_loop(0, nz, zbody, ())

    pltpu.sync_copy(src4.at[c, s], sidx)
    pltpu.sync_copy(dst4.at[c, s], didx)
    plsc.subcore_barrier()

    bufs = (bufa, bufb)
    sems = (sema, semb)

    for b in range(NBUF):
        pltpu.make_async_copy(x2.at[sidx.at[b]], bufs[b], sems[b]).start()

    def group_body(g, _):
        for b in range(NBUF):
            t = g * NBUF + b
            pltpu.make_async_copy(x2.at[sidx.at[t]], bufs[b], sems[b]).wait()
            nxt = t + NBUF

            @pl.when(nxt < NCHUNK)
            def _():
                pltpu.make_async_copy(
                    x2.at[sidx.at[nxt]], bufs[b], sems[b]).start()

            pltpu.sync_copy(bufs[b], accs.at[didx.at[t]], add=True)
        return ()
    lax.fori_loop(0, NCHUNK // NBUF, group_body, ())

    plsc.subcore_barrier()

    def wbody(t, _):
        r = start + t * ZROW
        pltpu.sync_copy(accs.at[pl.ds(r, ZROW)], agg_out.at[c, pl.ds(r, ZROW)])
        return ()
    lax.fori_loop(0, nz, wbody, ())


_sc_gin = pl.kernel(
    _sc_gin_body,
    out_type=jax.ShapeDtypeStruct((2, N, H), f32),
    mesh=plsc.VectorSubcoreMesh(core_axis_name="c", subcore_axis_name="s"),
    scratch_types=[
        pltpu.VMEM((NCHUNK, CHUNK), i32),
        pltpu.VMEM((NCHUNK, CHUNK), i32),
        pltpu.VMEM((CHUNK, H), f32),
        pltpu.VMEM((CHUNK, H), f32),
        pltpu.VMEM((ZROW, H), f32),
        pltpu.VMEM_SHARED((N, H), f32),
        pltpu.SemaphoreType.DMA,
        pltpu.SemaphoreType.DMA,
    ],
    compiler_params=pltpu.CompilerParams(use_tc_tiling_on_sc=False),
)


POOL_W = 3 * H
POOL_CHUNKS = N // CHUNK


def _sc_pool_body(gcat, batch, pool_out, bidx, buf, zb, accs, sem):
    c = lax.axis_index("c")
    s = lax.axis_index("s")

    _zero_vmem(zb, 8, POOL_W)
    pltpu.sync_copy(zb, accs.at[pl.ds(s * 8, 8)])
    plsc.subcore_barrier()

    def chunk_body(k, _):
        ch = s + k * NS

        @pl.when(ch < POOL_CHUNKS)
        def _():
            base = ch * CHUNK
            pltpu.sync_copy(batch.at[pl.ds(base, CHUNK)], bidx)
            pltpu.sync_copy(gcat.at[c, pl.ds(base, CHUNK)], buf)
            pltpu.sync_copy(buf, accs.at[bidx], add=True)
        return ()
    lax.fori_loop(0, (POOL_CHUNKS + NS - 1) // NS, chunk_body, ())

    plsc.subcore_barrier()
    pltpu.sync_copy(accs.at[pl.ds(s * 8, 8)],
                    pool_out.at[c, pl.ds(s * 8, 8)])


_sc_pool = pl.kernel(
    _sc_pool_body,
    out_type=jax.ShapeDtypeStruct((2, B, POOL_W), f32),
    mesh=plsc.VectorSubcoreMesh(core_axis_name="c", subcore_axis_name="s"),
    scratch_types=[
        pltpu.VMEM((CHUNK,), i32),
        pltpu.VMEM((CHUNK, POOL_W), f32),
        pltpu.VMEM((8, POOL_W), f32),
        pltpu.VMEM_SHARED((B, POOL_W), f32),
        pltpu.SemaphoreType.DMA,
    ],
    compiler_params=pltpu.CompilerParams(use_tc_tiling_on_sc=False),
)


def _bn(x, g, b, eps=1e-5):
    m = jnp.mean(x, axis=0)
    v = jnp.mean((x - m) ** 2, axis=0)
    return (x - m) / jnp.sqrt(v + eps) * g + b


def _tc_prep_body(xr, xp, wr, wp, avr, avp, h_ref, a_ref):
    for idx, (x, w, av) in enumerate(((xr, wr, avr), (xp, wp, avp))):
        h = jnp.dot(x[...], w[...], preferred_element_type=f32)
        h_ref[idx] = h
        a_s = jnp.sum(h * av[0], axis=1)
        a_d = jnp.sum(h * av[1], axis=1)
        a_ref[idx] = jnp.concatenate(
            [a_s[:, None], a_d[:, None], jnp.zeros((N, 6), f32)], axis=1)


_TC_PARAMS = pltpu.CompilerParams(vmem_limit_bytes=110 * 1024 * 1024)

_tc_prep = pl.pallas_call(
    _tc_prep_body,
    out_shape=[jax.ShapeDtypeStruct((2, N, H), f32),
               jax.ShapeDtypeStruct((2, N, 8), f32)],
    compiler_params=_TC_PARAMS,
)


def _tc_gatpost_body(acc, br, bp, gr, cbr, gp, cbp, g1_ref):
    for idx, (bb, g, cb) in enumerate(((br, gr, cbr), (bp, gp, cbp))):
        num = acc[idx, :, :H]
        den = acc[idx, :, H:H + 1]
        y = num / (den + 1e-16) + bb[...]
        g1_ref[idx] = jax.nn.relu(_bn(y, g[...], cb[...]))


_tc_gatpost = pl.pallas_call(
    _tc_gatpost_body,
    out_shape=jax.ShapeDtypeStruct((2, N, H), f32),
    compiler_params=_TC_PARAMS,
)


def _gin_mlp(x, agg, w1, b1, g1, be1, w2, b2, g2, be2):
    h = x + agg
    h = jnp.dot(h, w1[...], preferred_element_type=f32) + b1[...]
    h = jax.nn.relu(_bn(h, g1[...], be1[...]))
    h = jnp.dot(h, w2[...], preferred_element_type=f32) + b2[...]
    return jax.nn.relu(_bn(h, g2[...], be2[...]))


def _tc_gin_body(x, agg, *args):
    out_ref = args[-1]
    pr, pp = args[:8], args[8:16]
    for idx, p in enumerate((pr, pp)):
        out_ref[idx] = _gin_mlp(x[idx], agg[idx], *p)


_tc_gin = pl.pallas_call(
    _tc_gin_body,
    out_shape=jax.ShapeDtypeStruct((2, N, H), f32),
    compiler_params=_TC_PARAMS,
)


def _tc_gin_cat_body(x, agg, g1, *args):
    out_ref = args[-1]
    pr, pp = args[:8], args[8:16]
    for idx, p in enumerate((pr, pp)):
        g3 = _gin_mlp(x[idx], agg[idx], *p)
        out_ref[idx] = jnp.concatenate([g1[idx], x[idx], g3], axis=1)


_tc_gin_cat = pl.pallas_call(
    _tc_gin_cat_body,
    out_shape=jax.ShapeDtypeStruct((2, N, POOL_W), f32),
    compiler_params=_TC_PARAMS,
)


def _tc_head_body(pool, w1, b1, w2, b2, wc, bc, out_ref):
    conv = jnp.concatenate([pool[0], pool[1]], axis=1)
    z = jnp.dot(conv, w1[...], preferred_element_type=f32) + b1[...]
    z = jnp.dot(z, w2[...], preferred_element_type=f32) + b2[...]
    z = jax.nn.relu(z)
    out_ref[...] = jnp.dot(z, wc[...], preferred_element_type=f32) + bc[...]


_tc_head = pl.pallas_call(
    _tc_head_body,
    out_shape=jax.ShapeDtypeStruct((B, 10), f32),
    compiler_params=_TC_PARAMS,
)


def _gin_args(p):
    return (p["W1"], p["b1"][None, :], p["g1"][None, :], p["be1"][None, :],
            p["W2"], p["b2"][None, :], p["g2"][None, :], p["be2"][None, :])


def kernel(x_r, edge_r, x_p, edge_p, batch, params):
    pr, pp = params["gat_r"], params["gat_p"]
    src = jnp.concatenate([edge_r[0], edge_p[0] + N]).reshape(
        2, NS, NCHUNK, CHUNK)
    dst = jnp.concatenate([edge_r[1], edge_p[1]]).reshape(
        2, NS, NCHUNK, CHUNK)

    h, a8 = _tc_prep(x_r, x_p, pr["W"], pp["W"],
                     jnp.stack([pr["asrc"], pr["adst"]]),
                     jnp.stack([pp["asrc"], pp["adst"]]))
    h2 = h.reshape(2 * N, H)
    a2 = a8.reshape(2 * N, 8)
    acc = _sc_gat(h2, a2[:, 0], a2[:, 1], src, dst)

    g1 = _tc_gatpost(acc,
                     pr["b"][None, :], pp["b"][None, :],
                     params["bn_r"]["g"][None, :], params["bn_r"]["b"][None, :],
                     params["bn_p"]["g"][None, :], params["bn_p"]["b"][None, :])

    agg1 = _sc_gin(g1.reshape(2 * N, H), src, dst)
    g2 = _tc_gin(g1, agg1, *_gin_args(params["gin2_r"]),
                 *_gin_args(params["gin2_p"]))
    agg2 = _sc_gin(g2.reshape(2 * N, H), src, dst)
    gcat = _tc_gin_cat(g2, agg2, g1, *_gin_args(params["gin3_r"]),
                       *_gin_args(params["gin3_p"]))

    pool = _sc_pool(gcat, batch)

    fc = params["fc"]
    return _tc_head(pool, fc["W1"], fc["b1"][None, :],
                    fc["W2"], fc["b2"][None, :],
                    fc["Wc"], fc["bc"][None, :])


# speedup vs baseline: 1.5509x; 1.0794x over previous
import functools

import jax
import jax.numpy as jnp
from jax import lax
from jax.experimental import pallas as pl
from jax.experimental.pallas import tpu as pltpu
from jax.experimental.pallas import tpu_sc as plsc

N = 10000
E = 320000
B = 128
H = 64
NS = 16
CHUNK = 80
EPW = E // NS
NCHUNK = EPW // CHUNK
GAT_W = 80
NBUF = 2
ZROW = 40
SLAB = 125
NSLAB = NCHUNK // SLAB

f32 = jnp.float32
i32 = jnp.int32


def _tile_rows(s):
    start = s * 640
    nz = jnp.minimum(16, (N - start) // ZROW)
    return start, nz


def _zero_vmem(zb, nrow, ncol):
    zeros16 = jnp.zeros((16,), f32)

    def body(t, _):
        for j in range(ncol // 16):
            zb[t, pl.ds(j * 16, 16)] = zeros16
        return ()
    lax.fori_loop(0, nrow, body, ())


def _sc_gat_body(h2, as2, ad2, src4, dst4, acc_out,
                 asv, adv, sidx, didx, buf1a, buf1b, buf2,
                 exv, zb, accs, sema, semb):
    c = lax.axis_index("c")
    s = lax.axis_index("s")

    _zero_vmem(zb, ZROW, GAT_W)
    start, nz = _tile_rows(s)

    def zbody(t, _):
        pltpu.sync_copy(zb, accs.at[pl.ds(start + t * ZROW, ZROW)])
        return ()
    lax.fori_loop(0, nz, zbody, ())

    pltpu.sync_copy(as2, asv)
    pltpu.sync_copy(ad2, adv)
    plsc.subcore_barrier()

    off = c * N
    bufs1 = (buf1a, buf1b)
    sems = (sema, semb)

    def _weight_chunk(t, b1):
        for gg in range(CHUNK // 16):
            si = sidx[t, pl.ds(gg * 16, 16)]
            di = didx[t, pl.ds(gg * 16, 16)] + off
            e = plsc.load_gather(asv, [si]) + plsc.load_gather(adv, [di])
            e = jnp.maximum(e, 0.2 * e)
            exv[pl.ds(gg * 16, 16)] = jnp.exp(e)

        @plsc.parallel_loop(0, CHUNK, 1, unroll=10)
        def scale_body(i):
            w = plsc.load_gather(exv, [jnp.zeros((16,), i32) + i])
            for j in range(H // 16):
                buf2[i, pl.ds(j * 16, 16)] = b1[i, pl.ds(j * 16, 16)] * w
            buf2[i, pl.ds(H, 16)] = w

    for k in range(NSLAB):
        pltpu.sync_copy(src4.at[c, s, pl.ds(k * SLAB, SLAB)], sidx)
        pltpu.sync_copy(dst4.at[c, s, pl.ds(k * SLAB, SLAB)], didx)

        for b in range(NBUF):
            pltpu.make_async_copy(h2.at[sidx.at[b]], bufs1[b], sems[b]).start()

        def group_body(g, _):
            for b in range(NBUF):
                t = g * NBUF + b
                pltpu.make_async_copy(
                    h2.at[sidx.at[t]], bufs1[b], sems[b]).wait()
                _weight_chunk(t, bufs1[b])
                nxt = t + NBUF

                @pl.when(nxt < SLAB)
                def _():
                    pltpu.make_async_copy(
                        h2.at[sidx.at[nxt]], bufs1[b], sems[b]).start()

                pltpu.sync_copy(buf2, accs.at[didx.at[t]], add=True)
            return ()
        lax.fori_loop(0, SLAB // NBUF, group_body, ())

        for t in range(SLAB - SLAB % NBUF, SLAB):
            b = t % NBUF
            pltpu.make_async_copy(h2.at[sidx.at[t]], bufs1[b], sems[b]).wait()
            _weight_chunk(t, bufs1[b])
            pltpu.sync_copy(buf2, accs.at[didx.at[t]], add=True)

    plsc.subcore_barrier()

    def wbody(t, _):
        r = start + t * ZROW
        pltpu.sync_copy(accs.at[pl.ds(r, ZROW)], acc_out.at[c, pl.ds(r, ZROW)])
        return ()
    lax.fori_loop(0, nz, wbody, ())


_sc_gat = pl.kernel(
    _sc_gat_body,
    out_type=jax.ShapeDtypeStruct((2, N, GAT_W), f32),
    mesh=plsc.VectorSubcoreMesh(core_axis_name="c", subcore_axis_name="s"),
    scratch_types=[
        pltpu.VMEM((2 * N,), f32),
        pltpu.VMEM((2 * N,), f32),
        pltpu.VMEM((SLAB, CHUNK), i32),
        pltpu.VMEM((SLAB, CHUNK), i32),
        pltpu.VMEM((CHUNK, H), f32),
        pltpu.VMEM((CHUNK, H), f32),
        pltpu.VMEM((CHUNK, GAT_W), f32),
        pltpu.VMEM((CHUNK,), f32),
        pltpu.VMEM((ZROW, GAT_W), f32),
        pltpu.VMEM_SHARED((N, GAT_W), f32),
        pltpu.SemaphoreType.DMA,
        pltpu.SemaphoreType.DMA,
    ],
    compiler_params=pltpu.CompilerParams(needs_layout_passes=False,
                                         use_tc_tiling_on_sc=False),
)


def _sc_gin_body(x2, src4, dst4, agg_out,
                 sidx, didx, bufa, bufb, zb, accs, sema, semb):
    c = lax.axis_index("c")
    s = lax.axis_index("s")

    _zero_vmem(zb, ZROW, H)
    start, nz = _tile_rows(s)

    def zbody(t, _):
        pltpu.sync_copy(zb, accs.at[pl.ds(start + t * ZROW, ZROW)])
        return ()
    lax.fori_loop(0, nz, zbody, ())

    pltpu.sync_copy(src4.at[c, s], sidx)
    pltpu.sync_copy(dst4.at[c, s], didx)
    plsc.subcore_barrier()

    bufs = (bufa, bufb)
    sems = (sema, semb)

    for b in range(NBUF):
        pltpu.make_async_copy(x2.at[sidx.at[b]], bufs[b], sems[b]).start()

    def group_body(g, _):
        for b in range(NBUF):
            t = g * NBUF + b
            pltpu.make_async_copy(x2.at[sidx.at[t]], bufs[b], sems[b]).wait()
            nxt = t + NBUF

            @pl.when(nxt < NCHUNK)
            def _():
                pltpu.make_async_copy(
                    x2.at[sidx.at[nxt]], bufs[b], sems[b]).start()

            pltpu.sync_copy(bufs[b], accs.at[didx.at[t]], add=True)
        return ()
    lax.fori_loop(0, NCHUNK // NBUF, group_body, ())

    plsc.subcore_barrier()

    def wbody(t, _):
        r = start + t * ZROW
        pltpu.sync_copy(accs.at[pl.ds(r, ZROW)], agg_out.at[c, pl.ds(r, ZROW)])
        return ()
    lax.fori_loop(0, nz, wbody, ())


_sc_gin = pl.kernel(
    _sc_gin_body,
    out_type=jax.ShapeDtypeStruct((2, N, H), f32),
    mesh=plsc.VectorSubcoreMesh(core_axis_name="c", subcore_axis_name="s"),
    scratch_types=[
        pltpu.VMEM((NCHUNK, CHUNK), i32),
        pltpu.VMEM((NCHUNK, CHUNK), i32),
        pltpu.VMEM((CHUNK, H), f32),
        pltpu.VMEM((CHUNK, H), f32),
        pltpu.VMEM((ZROW, H), f32),
        pltpu.VMEM_SHARED((N, H), f32),
        pltpu.SemaphoreType.DMA,
        pltpu.SemaphoreType.DMA,
    ],
    compiler_params=pltpu.CompilerParams(use_tc_tiling_on_sc=False),
)


def _bn(x, g, b, eps=1e-5):
    m = jnp.mean(x, axis=0)
    v = jnp.mean((x - m) ** 2, axis=0)
    return (x - m) / jnp.sqrt(v + eps) * g + b


def _tc_prep_body(xr, xp, wr, wp, avr, avp, h_ref, a_ref):
    for idx, (x, w, av) in enumerate(((xr, wr, avr), (xp, wp, avp))):
        h = jnp.dot(x[...], w[...], preferred_element_type=f32)
        h_ref[idx] = h
        a_s = jnp.sum(h * av[0], axis=1)
        a_d = jnp.sum(h * av[1], axis=1)
        a_ref[idx] = jnp.concatenate(
            [a_s[:, None], a_d[:, None], jnp.zeros((N, 6), f32)], axis=1)


_TC_PARAMS = pltpu.CompilerParams(vmem_limit_bytes=110 * 1024 * 1024)

_tc_prep = pl.pallas_call(
    _tc_prep_body,
    out_shape=[jax.ShapeDtypeStruct((2, N, H), f32),
               jax.ShapeDtypeStruct((2, N, 8), f32)],
    compiler_params=_TC_PARAMS,
)


def _tc_gatpost_body(acc, br, bp, gr, cbr, gp, cbp, g1_ref):
    for idx, (bb, g, cb) in enumerate(((br, gr, cbr), (bp, gp, cbp))):
        num = acc[idx, :, :H]
        den = acc[idx, :, H:H + 1]
        y = num / (den + 1e-16) + bb[...]
        g1_ref[idx] = jax.nn.relu(_bn(y, g[...], cb[...]))


_tc_gatpost = pl.pallas_call(
    _tc_gatpost_body,
    out_shape=jax.ShapeDtypeStruct((2, N, H), f32),
    compiler_params=_TC_PARAMS,
)


def _gin_mlp(x, agg, w1, b1, g1, be1, w2, b2, g2, be2):
    h = x + agg
    h = jnp.dot(h, w1[...], preferred_element_type=f32) + b1[...]
    h = jax.nn.relu(_bn(h, g1[...], be1[...]))
    h = jnp.dot(h, w2[...], preferred_element_type=f32) + b2[...]
    return jax.nn.relu(_bn(h, g2[...], be2[...]))


def _tc_gin_body(x, agg, *args):
    out_ref = args[-1]
    pr, pp = args[:8], args[8:16]
    for idx, p in enumerate((pr, pp)):
        out_ref[idx] = _gin_mlp(x[idx], agg[idx], *p)


_tc_gin = pl.pallas_call(
    _tc_gin_body,
    out_shape=jax.ShapeDtypeStruct((2, N, H), f32),
    compiler_params=_TC_PARAMS,
)


def _tc_tail_body(x, agg, g1, batch, w1, b1, w2, b2, wc, bc, *args):
    out_ref = args[-1]
    pr, pp = args[:8], args[8:16]
    onehot = (batch[...][None, :]
              == lax.broadcasted_iota(i32, (B, N), 0)).astype(f32)
    pools = []
    for idx, p in enumerate((pr, pp)):
        g3 = _gin_mlp(x[idx], agg[idx], *p)
        gcat = jnp.concatenate([g1[idx], x[idx], g3], axis=1)
        pools.append(jnp.dot(onehot, gcat, preferred_element_type=f32))
    conv = jnp.concatenate(pools, axis=1)
    z = jnp.dot(conv, w1[...], preferred_element_type=f32) + b1[...]
    z = jnp.dot(z, w2[...], preferred_element_type=f32) + b2[...]
    z = jax.nn.relu(z)
    out_ref[...] = jnp.dot(z, wc[...], preferred_element_type=f32) + bc[...]


_tc_tail = pl.pallas_call(
    _tc_tail_body,
    out_shape=jax.ShapeDtypeStruct((B, 10), f32),
    compiler_params=_TC_PARAMS,
)


def _gin_args(p):
    return (p["W1"], p["b1"][None, :], p["g1"][None, :], p["be1"][None, :],
            p["W2"], p["b2"][None, :], p["g2"][None, :], p["be2"][None, :])


def kernel(x_r, edge_r, x_p, edge_p, batch, params):
    pr, pp = params["gat_r"], params["gat_p"]
    src = jnp.concatenate([edge_r[0], edge_p[0] + N]).reshape(
        2, NS, NCHUNK, CHUNK)
    dst = jnp.concatenate([edge_r[1], edge_p[1]]).reshape(
        2, NS, NCHUNK, CHUNK)

    h, a8 = _tc_prep(x_r, x_p, pr["W"], pp["W"],
                     jnp.stack([pr["asrc"], pr["adst"]]),
                     jnp.stack([pp["asrc"], pp["adst"]]))
    h2 = h.reshape(2 * N, H)
    a2 = a8.reshape(2 * N, 8)
    acc = _sc_gat(h2, a2[:, 0], a2[:, 1], src, dst)

    g1 = _tc_gatpost(acc,
                     pr["b"][None, :], pp["b"][None, :],
                     params["bn_r"]["g"][None, :], params["bn_r"]["b"][None, :],
                     params["bn_p"]["g"][None, :], params["bn_p"]["b"][None, :])

    agg1 = _sc_gin(g1.reshape(2 * N, H), src, dst)
    g2 = _tc_gin(g1, agg1, *_gin_args(params["gin2_r"]),
                 *_gin_args(params["gin2_p"]))
    agg2 = _sc_gin(g2.reshape(2 * N, H), src, dst)
    fc = params["fc"]
    return _tc_tail(g2, agg2, g1, batch,
                    fc["W1"], fc["b1"][None, :],
                    fc["W2"], fc["b2"][None, :],
                    fc["Wc"], fc["bc"][None, :],
                    *_gin_args(params["gin3_r"]),
                    *_gin_args(params["gin3_p"]))


# speedup vs baseline: 1.7185x; 1.1081x over previous
import functools

import jax
import jax.numpy as jnp
from jax import lax
from jax.experimental import pallas as pl
from jax.experimental.pallas import tpu as pltpu
from jax.experimental.pallas import tpu_sc as plsc

N = 10000
E = 320000
B = 128
H = 64
NS = 16
CHUNK = 80
EPW = E // NS
NCHUNK = EPW // CHUNK
GAT_W = 80
NBUF = 2
ZROW = 40
SLAB = 125
NSLAB = NCHUNK // SLAB

f32 = jnp.float32
i32 = jnp.int32


def _tile_rows(s):
    start = s * 640
    nz = jnp.minimum(16, (N - start) // ZROW)
    return start, nz


def _zero_vmem(zb, nrow, ncol):
    zeros16 = jnp.zeros((16,), f32)

    def body(t, _):
        for j in range(ncol // 16):
            zb[t, pl.ds(j * 16, 16)] = zeros16
        return ()
    lax.fori_loop(0, nrow, body, ())


def _sc_gat_body(h2, as2, ad2, src4, dst4, acc_out,
                 asv, adv, sidx, didx, buf1a, buf1b, buf2,
                 exv, zb, accs, sema, semb):
    c = lax.axis_index("c")
    s = lax.axis_index("s")

    _zero_vmem(zb, ZROW, GAT_W)
    start, nz = _tile_rows(s)

    def zbody(t, _):
        pltpu.sync_copy(zb, accs.at[pl.ds(start + t * ZROW, ZROW)])
        return ()
    lax.fori_loop(0, nz, zbody, ())

    pltpu.sync_copy(as2, asv)
    pltpu.sync_copy(ad2, adv)
    plsc.subcore_barrier()

    off = c * N
    bufs1 = (buf1a, buf1b)
    sems = (sema, semb)

    def _weight_chunk(t, b1):
        for gg in range(CHUNK // 16):
            si = sidx[t, pl.ds(gg * 16, 16)]
            di = didx[t, pl.ds(gg * 16, 16)] + off
            e = plsc.load_gather(asv, [si]) + plsc.load_gather(adv, [di])
            e = jnp.maximum(e, 0.2 * e)
            exv[pl.ds(gg * 16, 16)] = jnp.exp(e)

        @plsc.parallel_loop(0, CHUNK, 1, unroll=10)
        def scale_body(i):
            w = plsc.load_gather(exv, [jnp.zeros((16,), i32) + i])
            for j in range(H // 16):
                buf2[i, pl.ds(j * 16, 16)] = b1[i, pl.ds(j * 16, 16)] * w
            buf2[i, pl.ds(H, 16)] = w

    for k in range(NSLAB):
        pltpu.sync_copy(src4.at[c, s, pl.ds(k * SLAB, SLAB)], sidx)
        pltpu.sync_copy(dst4.at[c, s, pl.ds(k * SLAB, SLAB)], didx)

        for b in range(NBUF):
            pltpu.make_async_copy(h2.at[sidx.at[b]], bufs1[b], sems[b]).start()

        def group_body(g, _):
            for b in range(NBUF):
                t = g * NBUF + b
                pltpu.make_async_copy(
                    h2.at[sidx.at[t]], bufs1[b], sems[b]).wait()
                _weight_chunk(t, bufs1[b])
                nxt = t + NBUF

                @pl.when(nxt < SLAB)
                def _():
                    pltpu.make_async_copy(
                        h2.at[sidx.at[nxt]], bufs1[b], sems[b]).start()

                pltpu.sync_copy(buf2, accs.at[didx.at[t]], add=True)
            return ()
        lax.fori_loop(0, SLAB // NBUF, group_body, ())

        for t in range(SLAB - SLAB % NBUF, SLAB):
            b = t % NBUF
            pltpu.make_async_copy(h2.at[sidx.at[t]], bufs1[b], sems[b]).wait()
            _weight_chunk(t, bufs1[b])
            pltpu.sync_copy(buf2, accs.at[didx.at[t]], add=True)

    plsc.subcore_barrier()

    def wbody(t, _):
        r = start + t * ZROW
        pltpu.sync_copy(accs.at[pl.ds(r, ZROW)], acc_out.at[c, pl.ds(r, ZROW)])
        return ()
    lax.fori_loop(0, nz, wbody, ())


_sc_gat = pl.kernel(
    _sc_gat_body,
    out_type=jax.ShapeDtypeStruct((2, N, GAT_W), f32),
    mesh=plsc.VectorSubcoreMesh(core_axis_name="c", subcore_axis_name="s"),
    scratch_types=[
        pltpu.VMEM((2 * N,), f32),
        pltpu.VMEM((2 * N,), f32),
        pltpu.VMEM((SLAB, CHUNK), i32),
        pltpu.VMEM((SLAB, CHUNK), i32),
        pltpu.VMEM((CHUNK, H), f32),
        pltpu.VMEM((CHUNK, H), f32),
        pltpu.VMEM((CHUNK, GAT_W), f32),
        pltpu.VMEM((CHUNK,), f32),
        pltpu.VMEM((ZROW, GAT_W), f32),
        pltpu.VMEM_SHARED((N, GAT_W), f32),
        pltpu.SemaphoreType.DMA,
        pltpu.SemaphoreType.DMA,
    ],
    compiler_params=pltpu.CompilerParams(needs_layout_passes=False,
                                         use_tc_tiling_on_sc=False),
)


GBUF = 4


def _sc_gin_body(x2, src4, dst4, agg_out,
                 sidx, didx, bufa, bufb, bufc, bufd, zb, accs,
                 sema, semb, semc, semd):
    c = lax.axis_index("c")
    s = lax.axis_index("s")

    _zero_vmem(zb, ZROW, H)
    start, nz = _tile_rows(s)

    def zbody(t, _):
        pltpu.sync_copy(zb, accs.at[pl.ds(start + t * ZROW, ZROW)])
        return ()
    lax.fori_loop(0, nz, zbody, ())

    pltpu.sync_copy(src4.at[c, s], sidx)
    pltpu.sync_copy(dst4.at[c, s], didx)
    plsc.subcore_barrier()

    bufs = (bufa, bufb, bufc, bufd)
    sems = (sema, semb, semc, semd)

    for b in range(GBUF):
        pltpu.make_async_copy(x2.at[sidx.at[b]], bufs[b], sems[b]).start()

    def group_body(g, _):
        for b in range(GBUF):
            t = g * GBUF + b
            pltpu.make_async_copy(x2.at[sidx.at[t]], bufs[b], sems[b]).wait()
            nxt = t + GBUF

            @pl.when(nxt < NCHUNK)
            def _():
                pltpu.make_async_copy(
                    x2.at[sidx.at[nxt]], bufs[b], sems[b]).start()

            pltpu.sync_copy(bufs[b], accs.at[didx.at[t]], add=True)
        return ()
    lax.fori_loop(0, NCHUNK // GBUF, group_body, ())

    for t in range(NCHUNK - NCHUNK % GBUF, NCHUNK):
        b = t % GBUF
        pltpu.make_async_copy(x2.at[sidx.at[t]], bufs[b], sems[b]).wait()
        pltpu.sync_copy(bufs[b], accs.at[didx.at[t]], add=True)

    plsc.subcore_barrier()

    def wbody(t, _):
        r = start + t * ZROW
        pltpu.sync_copy(accs.at[pl.ds(r, ZROW)], agg_out.at[c, pl.ds(r, ZROW)])
        return ()
    lax.fori_loop(0, nz, wbody, ())


_sc_gin = pl.kernel(
    _sc_gin_body,
    out_type=jax.ShapeDtypeStruct((2, N, H), f32),
    mesh=plsc.VectorSubcoreMesh(core_axis_name="c", subcore_axis_name="s"),
    scratch_types=[
        pltpu.VMEM((NCHUNK, CHUNK), i32),
        pltpu.VMEM((NCHUNK, CHUNK), i32),
        pltpu.VMEM((CHUNK, H), f32),
        pltpu.VMEM((CHUNK, H), f32),
        pltpu.VMEM((CHUNK, H), f32),
        pltpu.VMEM((CHUNK, H), f32),
        pltpu.VMEM((ZROW, H), f32),
        pltpu.VMEM_SHARED((N, H), f32),
        pltpu.SemaphoreType.DMA,
        pltpu.SemaphoreType.DMA,
        pltpu.SemaphoreType.DMA,
        pltpu.SemaphoreType.DMA,
    ],
    compiler_params=pltpu.CompilerParams(use_tc_tiling_on_sc=False),
)


def _bn(x, g, b, eps=1e-5):
    m = jnp.mean(x, axis=0)
    v = jnp.mean((x - m) ** 2, axis=0)
    return (x - m) / jnp.sqrt(v + eps) * g + b


def _tc_prep_body(xr, xp, wr, wp, avr, avp, h_ref, a_ref):
    for idx, (x, w, av) in enumerate(((xr, wr, avr), (xp, wp, avp))):
        h = jnp.dot(x[...], w[...], preferred_element_type=f32)
        h_ref[idx] = h
        a_s = jnp.sum(h * av[0], axis=1)
        a_d = jnp.sum(h * av[1], axis=1)
        a_ref[idx] = jnp.concatenate(
            [a_s[:, None], a_d[:, None], jnp.zeros((N, 6), f32)], axis=1)


_TC_PARAMS = pltpu.CompilerParams(vmem_limit_bytes=110 * 1024 * 1024)

_tc_prep = pl.pallas_call(
    _tc_prep_body,
    out_shape=[jax.ShapeDtypeStruct((2, N, H), f32),
               jax.ShapeDtypeStruct((2, N, 8), f32)],
    compiler_params=_TC_PARAMS,
)


def _tc_gatpost_body(acc, br, bp, gr, cbr, gp, cbp, g1_ref):
    for idx, (bb, g, cb) in enumerate(((br, gr, cbr), (bp, gp, cbp))):
        num = acc[idx, :, :H]
        den = acc[idx, :, H:H + 1]
        y = num / (den + 1e-16) + bb[...]
        g1_ref[idx] = jax.nn.relu(_bn(y, g[...], cb[...]))


_tc_gatpost = pl.pallas_call(
    _tc_gatpost_body,
    out_shape=jax.ShapeDtypeStruct((2, N, H), f32),
    compiler_params=_TC_PARAMS,
)


def _gin_mlp(x, agg, w1, b1, g1, be1, w2, b2, g2, be2):
    h = x + agg
    h = jnp.dot(h, w1[...], preferred_element_type=f32) + b1[...]
    h = jax.nn.relu(_bn(h, g1[...], be1[...]))
    h = jnp.dot(h, w2[...], preferred_element_type=f32) + b2[...]
    return jax.nn.relu(_bn(h, g2[...], be2[...]))


def _tc_gin_body(x, agg, *args):
    out_ref = args[-1]
    pr, pp = args[:8], args[8:16]
    for idx, p in enumerate((pr, pp)):
        out_ref[idx] = _gin_mlp(x[idx], agg[idx], *p)


_tc_gin = pl.pallas_call(
    _tc_gin_body,
    out_shape=jax.ShapeDtypeStruct((2, N, H), f32),
    compiler_params=_TC_PARAMS,
)


def _tc_tail_body(x, agg, g1, batch, w1, b1, w2, b2, wc, bc, *args):
    out_ref = args[-1]
    pr, pp = args[:8], args[8:16]
    onehot = (batch[...][None, :]
              == lax.broadcasted_iota(i32, (B, N), 0)).astype(f32)
    pools = []
    for idx, p in enumerate((pr, pp)):
        g3 = _gin_mlp(x[idx], agg[idx], *p)
        gcat = jnp.concatenate([g1[idx], x[idx], g3], axis=1)
        pools.append(jnp.dot(onehot, gcat, preferred_element_type=f32))
    conv = jnp.concatenate(pools, axis=1)
    z = jnp.dot(conv, w1[...], preferred_element_type=f32) + b1[...]
    z = jnp.dot(z, w2[...], preferred_element_type=f32) + b2[...]
    z = jax.nn.relu(z)
    out_ref[...] = jnp.dot(z, wc[...], preferred_element_type=f32) + bc[...]


_tc_tail = pl.pallas_call(
    _tc_tail_body,
    out_shape=jax.ShapeDtypeStruct((B, 10), f32),
    compiler_params=_TC_PARAMS,
)


def _gin_args(p):
    return (p["W1"], p["b1"][None, :], p["g1"][None, :], p["be1"][None, :],
            p["W2"], p["b2"][None, :], p["g2"][None, :], p["be2"][None, :])


def kernel(x_r, edge_r, x_p, edge_p, batch, params):
    pr, pp = params["gat_r"], params["gat_p"]
    src = jnp.concatenate([edge_r[0], edge_p[0] + N]).reshape(
        2, NS, NCHUNK, CHUNK)
    dst = jnp.concatenate([edge_r[1], edge_p[1]]).reshape(
        2, NS, NCHUNK, CHUNK)

    h, a8 = _tc_prep(x_r, x_p, pr["W"], pp["W"],
                     jnp.stack([pr["asrc"], pr["adst"]]),
                     jnp.stack([pp["asrc"], pp["adst"]]))
    h2 = h.reshape(2 * N, H)
    a2 = a8.reshape(2 * N, 8)
    acc = _sc_gat(h2, a2[:, 0], a2[:, 1], src, dst)

    g1 = _tc_gatpost(acc,
                     pr["b"][None, :], pp["b"][None, :],
                     params["bn_r"]["g"][None, :], params["bn_r"]["b"][None, :],
                     params["bn_p"]["g"][None, :], params["bn_p"]["b"][None, :])

    agg1 = _sc_gin(g1.reshape(2 * N, H), src, dst)
    g2 = _tc_gin(g1, agg1, *_gin_args(params["gin2_r"]),
                 *_gin_args(params["gin2_p"]))
    agg2 = _sc_gin(g2.reshape(2 * N, H), src, dst)
    fc = params["fc"]
    return _tc_tail(g2, agg2, g1, batch,
                    fc["W1"], fc["b1"][None, :],
                    fc["W2"], fc["b2"][None, :],
                    fc["Wc"], fc["bc"][None, :],
                    *_gin_args(params["gin3_r"]),
                    *_gin_args(params["gin3_p"]))


# speedup vs baseline: 1.7263x; 1.0045x over previous
import functools

import jax
import jax.numpy as jnp
from jax import lax
from jax.experimental import pallas as pl
from jax.experimental.pallas import tpu as pltpu
from jax.experimental.pallas import tpu_sc as plsc

N = 10000
E = 320000
B = 128
H = 64
NS = 16
CHUNK = 80
EPW = E // NS
NCHUNK = EPW // CHUNK
GAT_W = 72
NBUF = 3
ZROW = 40
SLAB = 125
NSLAB = NCHUNK // SLAB

f32 = jnp.float32
i32 = jnp.int32


def _tile_rows(s):
    start = s * 640
    nz = jnp.minimum(16, (N - start) // ZROW)
    return start, nz


def _zero_vmem(zb, nrow, ncol):
    zeros16 = jnp.zeros((16,), f32)

    def body(t, _):
        for j in range(ncol // 16):
            zb[t, pl.ds(j * 16, 16)] = zeros16
        return ()
    lax.fori_loop(0, nrow, body, ())


def _sc_gat_body(h2, as2, ad2, src4, dst4, acc_out,
                 asv, adv, sidx, didx, buf1a, buf1b, buf1c, buf2,
                 exv, zb, accs, sema, semb, semc):
    c = lax.axis_index("c")
    s = lax.axis_index("s")

    _zero_vmem(zb, ZROW, GAT_W)
    start, nz = _tile_rows(s)

    def zbody(t, _):
        pltpu.sync_copy(zb, accs.at[pl.ds(start + t * ZROW, ZROW)])
        return ()
    lax.fori_loop(0, nz, zbody, ())

    pltpu.sync_copy(as2, asv)
    pltpu.sync_copy(ad2, adv)
    plsc.subcore_barrier()

    off = c * N
    bufs1 = (buf1a, buf1b, buf1c)
    sems = (sema, semb, semc)
    lane8 = H + jnp.bitwise_and(lax.iota(i32, 16), 7)

    def _weight_chunk(t, b1):
        for gg in range(CHUNK // 16):
            si = sidx[t, pl.ds(gg * 16, 16)]
            di = didx[t, pl.ds(gg * 16, 16)] + off
            e = plsc.load_gather(asv, [si]) + plsc.load_gather(adv, [di])
            e = jnp.maximum(e, 0.2 * e)
            exv[pl.ds(gg * 16, 16)] = jnp.exp(e)

        @plsc.parallel_loop(0, CHUNK, 1, unroll=10)
        def scale_body(i):
            w = plsc.load_gather(exv, [jnp.zeros((16,), i32) + i])
            for j in range(H // 16):
                buf2[i, pl.ds(j * 16, 16)] = b1[i, pl.ds(j * 16, 16)] * w
            plsc.store_scatter(buf2, [jnp.zeros((16,), i32) + i, lane8], w)

    for k in range(NSLAB):
        pltpu.sync_copy(src4.at[c, s, pl.ds(k * SLAB, SLAB)], sidx)
        pltpu.sync_copy(dst4.at[c, s, pl.ds(k * SLAB, SLAB)], didx)

        for b in range(NBUF):
            pltpu.make_async_copy(h2.at[sidx.at[b]], bufs1[b], sems[b]).start()

        def group_body(g, _):
            for b in range(NBUF):
                t = g * NBUF + b
                pltpu.make_async_copy(
                    h2.at[sidx.at[t]], bufs1[b], sems[b]).wait()
                _weight_chunk(t, bufs1[b])
                nxt = t + NBUF

                @pl.when(nxt < SLAB)
                def _():
                    pltpu.make_async_copy(
                        h2.at[sidx.at[nxt]], bufs1[b], sems[b]).start()

                pltpu.sync_copy(buf2, accs.at[didx.at[t]], add=True)
            return ()
        lax.fori_loop(0, SLAB // NBUF, group_body, ())

        for t in range(SLAB - SLAB % NBUF, SLAB):
            b = t % NBUF
            pltpu.make_async_copy(h2.at[sidx.at[t]], bufs1[b], sems[b]).wait()
            _weight_chunk(t, bufs1[b])
            pltpu.sync_copy(buf2, accs.at[didx.at[t]], add=True)

    plsc.subcore_barrier()

    def wbody(t, _):
        r = start + t * ZROW
        pltpu.sync_copy(accs.at[pl.ds(r, ZROW)], acc_out.at[c, pl.ds(r, ZROW)])
        return ()
    lax.fori_loop(0, nz, wbody, ())


_sc_gat = pl.kernel(
    _sc_gat_body,
    out_type=jax.ShapeDtypeStruct((2, N, GAT_W), f32),
    mesh=plsc.VectorSubcoreMesh(core_axis_name="c", subcore_axis_name="s"),
    scratch_types=[
        pltpu.VMEM((2 * N,), f32),
        pltpu.VMEM((2 * N,), f32),
        pltpu.VMEM((SLAB, CHUNK), i32),
        pltpu.VMEM((SLAB, CHUNK), i32),
        pltpu.VMEM((CHUNK, H), f32),
        pltpu.VMEM((CHUNK, H), f32),
        pltpu.VMEM((CHUNK, H), f32),
        pltpu.VMEM((CHUNK, GAT_W), f32),
        pltpu.VMEM((CHUNK,), f32),
        pltpu.VMEM((ZROW, GAT_W), f32),
        pltpu.VMEM_SHARED((N, GAT_W), f32),
        pltpu.SemaphoreType.DMA,
        pltpu.SemaphoreType.DMA,
        pltpu.SemaphoreType.DMA,
    ],
    compiler_params=pltpu.CompilerParams(needs_layout_passes=False,
                                         use_tc_tiling_on_sc=False),
)


GBUF = 4


def _sc_gin_body(x2, src4, dst4, agg_out,
                 sidx, didx, bufa, bufb, bufc, bufd, zb, accs,
                 sema, semb, semc, semd):
    c = lax.axis_index("c")
    s = lax.axis_index("s")

    _zero_vmem(zb, ZROW, H)
    start, nz = _tile_rows(s)

    def zbody(t, _):
        pltpu.sync_copy(zb, accs.at[pl.ds(start + t * ZROW, ZROW)])
        return ()
    lax.fori_loop(0, nz, zbody, ())

    pltpu.sync_copy(src4.at[c, s], sidx)
    pltpu.sync_copy(dst4.at[c, s], didx)
    plsc.subcore_barrier()

    bufs = (bufa, bufb, bufc, bufd)
    sems = (sema, semb, semc, semd)

    for b in range(GBUF):
        pltpu.make_async_copy(x2.at[sidx.at[b]], bufs[b], sems[b]).start()

    def group_body(g, _):
        for b in range(GBUF):
            t = g * GBUF + b
            pltpu.make_async_copy(x2.at[sidx.at[t]], bufs[b], sems[b]).wait()
            nxt = t + GBUF

            @pl.when(nxt < NCHUNK)
            def _():
                pltpu.make_async_copy(
                    x2.at[sidx.at[nxt]], bufs[b], sems[b]).start()

            pltpu.sync_copy(bufs[b], accs.at[didx.at[t]], add=True)
        return ()
    lax.fori_loop(0, NCHUNK // GBUF, group_body, ())

    for t in range(NCHUNK - NCHUNK % GBUF, NCHUNK):
        b = t % GBUF
        pltpu.make_async_copy(x2.at[sidx.at[t]], bufs[b], sems[b]).wait()
        pltpu.sync_copy(bufs[b], accs.at[didx.at[t]], add=True)

    plsc.subcore_barrier()

    def wbody(t, _):
        r = start + t * ZROW
        pltpu.sync_copy(accs.at[pl.ds(r, ZROW)], agg_out.at[c, pl.ds(r, ZROW)])
        return ()
    lax.fori_loop(0, nz, wbody, ())


_sc_gin = pl.kernel(
    _sc_gin_body,
    out_type=jax.ShapeDtypeStruct((2, N, H), f32),
    mesh=plsc.VectorSubcoreMesh(core_axis_name="c", subcore_axis_name="s"),
    scratch_types=[
        pltpu.VMEM((NCHUNK, CHUNK), i32),
        pltpu.VMEM((NCHUNK, CHUNK), i32),
        pltpu.VMEM((CHUNK, H), f32),
        pltpu.VMEM((CHUNK, H), f32),
        pltpu.VMEM((CHUNK, H), f32),
        pltpu.VMEM((CHUNK, H), f32),
        pltpu.VMEM((ZROW, H), f32),
        pltpu.VMEM_SHARED((N, H), f32),
        pltpu.SemaphoreType.DMA,
        pltpu.SemaphoreType.DMA,
        pltpu.SemaphoreType.DMA,
        pltpu.SemaphoreType.DMA,
    ],
    compiler_params=pltpu.CompilerParams(use_tc_tiling_on_sc=False),
)


def _bn(x, g, b, eps=1e-5):
    m = jnp.mean(x, axis=0)
    v = jnp.mean((x - m) ** 2, axis=0)
    return (x - m) / jnp.sqrt(v + eps) * g + b


def _tc_prep_body(xr, xp, wr, wp, avr, avp, h_ref, a_ref):
    for idx, (x, w, av) in enumerate(((xr, wr, avr), (xp, wp, avp))):
        h = jnp.dot(x[...], w[...], preferred_element_type=f32)
        h_ref[idx] = h
        a_s = jnp.sum(h * av[0], axis=1)
        a_d = jnp.sum(h * av[1], axis=1)
        a_ref[idx] = jnp.concatenate(
            [a_s[:, None], a_d[:, None], jnp.zeros((N, 6), f32)], axis=1)


_TC_PARAMS = pltpu.CompilerParams(vmem_limit_bytes=110 * 1024 * 1024)

_tc_prep = pl.pallas_call(
    _tc_prep_body,
    out_shape=[jax.ShapeDtypeStruct((2, N, H), f32),
               jax.ShapeDtypeStruct((2, N, 8), f32)],
    compiler_params=_TC_PARAMS,
)


def _tc_gatpost_body(acc, br, bp, gr, cbr, gp, cbp, g1_ref):
    for idx, (bb, g, cb) in enumerate(((br, gr, cbr), (bp, gp, cbp))):
        num = acc[idx, :, :H]
        den = acc[idx, :, H:H + 1]
        y = num / (den + 1e-16) + bb[...]
        g1_ref[idx] = jax.nn.relu(_bn(y, g[...], cb[...]))


_tc_gatpost = pl.pallas_call(
    _tc_gatpost_body,
    out_shape=jax.ShapeDtypeStruct((2, N, H), f32),
    compiler_params=_TC_PARAMS,
)


def _gin_mlp(x, agg, w1, b1, g1, be1, w2, b2, g2, be2):
    h = x + agg
    h = jnp.dot(h, w1[...], preferred_element_type=f32) + b1[...]
    h = jax.nn.relu(_bn(h, g1[...], be1[...]))
    h = jnp.dot(h, w2[...], preferred_element_type=f32) + b2[...]
    return jax.nn.relu(_bn(h, g2[...], be2[...]))


def _tc_gin_body(x, agg, *args):
    out_ref = args[-1]
    pr, pp = args[:8], args[8:16]
    for idx, p in enumerate((pr, pp)):
        out_ref[idx] = _gin_mlp(x[idx], agg[idx], *p)


_tc_gin = pl.pallas_call(
    _tc_gin_body,
    out_shape=jax.ShapeDtypeStruct((2, N, H), f32),
    compiler_params=_TC_PARAMS,
)


def _tc_tail_body(x, agg, g1, batch, w1, b1, w2, b2, wc, bc, *args):
    out_ref = args[-1]
    pr, pp = args[:8], args[8:16]
    onehot = (batch[...][None, :]
              == lax.broadcasted_iota(i32, (B, N), 0)).astype(f32)
    pools = []
    for idx, p in enumerate((pr, pp)):
        g3 = _gin_mlp(x[idx], agg[idx], *p)
        gcat = jnp.concatenate([g1[idx], x[idx], g3], axis=1)
        pools.append(jnp.dot(onehot, gcat, preferred_element_type=f32))
    conv = jnp.concatenate(pools, axis=1)
    z = jnp.dot(conv, w1[...], preferred_element_type=f32) + b1[...]
    z = jnp.dot(z, w2[...], preferred_element_type=f32) + b2[...]
    z = jax.nn.relu(z)
    out_ref[...] = jnp.dot(z, wc[...], preferred_element_type=f32) + bc[...]


_tc_tail = pl.pallas_call(
    _tc_tail_body,
    out_shape=jax.ShapeDtypeStruct((B, 10), f32),
    compiler_params=_TC_PARAMS,
)


def _gin_args(p):
    return (p["W1"], p["b1"][None, :], p["g1"][None, :], p["be1"][None, :],
            p["W2"], p["b2"][None, :], p["g2"][None, :], p["be2"][None, :])


def kernel(x_r, edge_r, x_p, edge_p, batch, params):
    pr, pp = params["gat_r"], params["gat_p"]
    src = jnp.concatenate([edge_r[0], edge_p[0] + N]).reshape(
        2, NS, NCHUNK, CHUNK)
    dst = jnp.concatenate([edge_r[1], edge_p[1]]).reshape(
        2, NS, NCHUNK, CHUNK)

    h, a8 = _tc_prep(x_r, x_p, pr["W"], pp["W"],
                     jnp.stack([pr["asrc"], pr["adst"]]),
                     jnp.stack([pp["asrc"], pp["adst"]]))
    h2 = h.reshape(2 * N, H)
    a2 = a8.reshape(2 * N, 8)
    acc = _sc_gat(h2, a2[:, 0], a2[:, 1], src, dst)

    g1 = _tc_gatpost(acc,
                     pr["b"][None, :], pp["b"][None, :],
                     params["bn_r"]["g"][None, :], params["bn_r"]["b"][None, :],
                     params["bn_p"]["g"][None, :], params["bn_p"]["b"][None, :])

    agg1 = _sc_gin(g1.reshape(2 * N, H), src, dst)
    g2 = _tc_gin(g1, agg1, *_gin_args(params["gin2_r"]),
                 *_gin_args(params["gin2_p"]))
    agg2 = _sc_gin(g2.reshape(2 * N, H), src, dst)
    fc = params["fc"]
    return _tc_tail(g2, agg2, g1, batch,
                    fc["W1"], fc["b1"][None, :],
                    fc["W2"], fc["b2"][None, :],
                    fc["Wc"], fc["bc"][None, :],
                    *_gin_args(params["gin3_r"]),
                    *_gin_args(params["gin3_p"]))
